# Initial kernel scaffold; baseline (speedup 1.0000x reference)
import jax
import jax.numpy as jnp
from jax.experimental import pallas as pl


def kernel(node_type, pressure, target_pressure, mesh_pos, world_pos, senders, receivers, is_trainning, params):
    raise NotImplementedError("write your pallas kernel here")


# speedup vs baseline: 3.0060x; 3.0060x over previous
import functools

import jax
import jax.numpy as jnp
from jax import lax
from jax.experimental import pallas as pl
from jax.experimental.pallas import tpu as pltpu
from jax.experimental.pallas import tpu_sc as plsc

F32 = jnp.float32

BE = 2000
BN = 2000


def _ln(x, g, b):
    mu = jnp.mean(x, axis=1, keepdims=True)
    var = jnp.mean((x - mu) * (x - mu), axis=1, keepdims=True)
    return (x - mu) / jnp.sqrt(var + 1e-5) * g + b


def _relu(x):
    return jnp.maximum(x, 0.0)


def _dot(x, w):
    return jnp.dot(x, w, preferred_element_type=F32)


def _full(shape):
    return pl.BlockSpec(shape, lambda i: (0,) * len(shape))


def _rows(block, minor=128):
    return pl.BlockSpec((block, minor), lambda i: (i, 0))


def _encoder_body(f_ref, w1, w2, w3, aux, wpq, out, p_out, q_out):
    a = aux[...]
    x = _relu(_dot(f_ref[...], w1[...]) + a[0:1])
    x = _relu(_dot(x, w2[...]) + a[1:2])
    x = _dot(x, w3[...]) + a[2:3]
    x = _ln(x, a[3:4], a[4:5])
    out[...] = x
    pq = _dot(x, wpq[...])
    p_out[...] = pq[:, :128] + a[5:6]
    q_out[...] = pq[:, 128:]


def _edge_enc_body(f_ref, w1, w2, w3, aux, out):
    a = aux[...]
    x = _relu(_dot(f_ref[...], w1[...]) + a[0:1])
    x = _relu(_dot(x, w2[...]) + a[1:2])
    x = _dot(x, w3[...]) + a[2:3]
    out[...] = _ln(x, a[3:4], a[4:5])


def _edge_step_body(gs, gr, el, w1c, w2, w3, aux, out):
    a = aux[...]
    elv = el[...]
    x = _relu(gs[...] + gr[...] + _dot(elv, w1c[...]))
    x = _relu(_dot(x, w2[...]) + a[0:1])
    x = _dot(x, w3[...]) + a[1:2]
    out[...] = _ln(x, a[2:3], a[3:4]) + elv


def _node_step_body(nl, a0, a1, v1, v2, v3, aux, wpq, out, p_out, q_out):
    a = aux[...]
    nlv = nl[...]
    x = jnp.concatenate([nlv, a0[...] + a1[...]], axis=1)
    x = _relu(_dot(x, v1[...]) + a[0:1])
    x = _relu(_dot(x, v2[...]) + a[1:2])
    x = _dot(x, v3[...]) + a[2:3]
    x = _ln(x, a[3:4], a[4:5]) + nlv
    out[...] = x
    pq = _dot(x, wpq[...])
    p_out[...] = pq[:, :128] + a[5:6]
    q_out[...] = pq[:, 128:]


def _node_last_body(nl, a0, a1, v1, v2, v3, aux, d1, d2, d3, daux, out):
    a = aux[...]
    da = daux[...]
    nlv = nl[...]
    x = jnp.concatenate([nlv, a0[...] + a1[...]], axis=1)
    x = _relu(_dot(x, v1[...]) + a[0:1])
    x = _relu(_dot(x, v2[...]) + a[1:2])
    x = _dot(x, v3[...]) + a[2:3]
    x = _ln(x, a[3:4], a[4:5]) + nlv
    y = _relu(_dot(x, d1[...]) + da[0:1])
    y = _relu(_dot(y, d2[...]) + da[1:2])
    y = _dot(y, d3[...]) + da[2:3, :3]
    out[...] = y


def _pad8(rows):
    out = jnp.zeros((8, 128), F32)
    for i, r in enumerate(rows):
        out = out.at[i, : r.shape[0]].set(r)
    return out


def _call_tc(body, grid, in_arrays, in_specs, out_shapes, out_specs):
    return pl.pallas_call(
        body,
        grid=(grid,),
        in_specs=in_specs,
        out_specs=out_specs,
        out_shape=out_shapes,
        compiler_params=pltpu.CompilerParams(
            dimension_semantics=("arbitrary",),
        ),
    )(*in_arrays)


_NC = 2
_NS = 16
_NW = _NC * _NS
_CH = 128


def _gather_pq(p_tab, q_tab, senders, receivers):
    e, d = p_tab.shape[0], p_tab.shape[1]
    e_edges = senders.shape[0]
    nchunks = e_edges // _CH
    maxiter = (nchunks + _NW - 1) // _NW
    mesh = plsc.VectorSubcoreMesh(core_axis_name="c", subcore_axis_name="s")

    @functools.partial(
        pl.kernel, mesh=mesh,
        out_type=(jax.ShapeDtypeStruct((e_edges, d), p_tab.dtype),
                  jax.ShapeDtypeStruct((e_edges, d), q_tab.dtype)),
        scratch_types=[
            pltpu.VMEM((_CH,), jnp.int32), pltpu.VMEM((_CH,), jnp.int32),
            pltpu.VMEM((_CH, d), p_tab.dtype), pltpu.VMEM((_CH, d), q_tab.dtype),
            pltpu.SemaphoreType.DMA, pltpu.SemaphoreType.DMA,
        ])
    def gk(p_hbm, q_hbm, s_hbm, r_hbm, op_hbm, oq_hbm, sidx, ridx, prow, qrow,
           sem1, sem2):
        wid = lax.axis_index("c") * _NS + lax.axis_index("s")

        def body(j, carry):
            c = wid + j * _NW

            @pl.when(c < nchunks)
            def _():
                base = c * _CH
                pltpu.sync_copy(s_hbm.at[pl.ds(base, _CH)], sidx)
                pltpu.sync_copy(r_hbm.at[pl.ds(base, _CH)], ridx)
                cp1 = pltpu.async_copy(p_hbm.at[sidx], prow, sem1)
                cp2 = pltpu.async_copy(q_hbm.at[ridx], qrow, sem2)
                cp1.wait()
                cp2.wait()
                pltpu.sync_copy(prow, op_hbm.at[pl.ds(base, _CH)])
                pltpu.sync_copy(qrow, oq_hbm.at[pl.ds(base, _CH)])
            return carry

        lax.fori_loop(0, maxiter, body, 0)

    return gk(p_tab, q_tab, senders, receivers)


def _scatter_partials(el, receivers, n):
    e_edges, d = el.shape
    nchunks = e_edges // _CH
    maxiter = (nchunks + _NW - 1) // _NW
    rpt = (n // (8 * _NS)) * 8
    rem = n - rpt * _NS
    mesh = plsc.VectorSubcoreMesh(core_axis_name="c", subcore_axis_name="s")

    @functools.partial(
        pl.kernel, mesh=mesh,
        out_type=jax.ShapeDtypeStruct((_NC, n, d), F32),
        scratch_types=[
            pltpu.VMEM((_CH,), jnp.int32),
            pltpu.VMEM((_CH, d), F32),
            pltpu.VMEM_SHARED((n, d), F32),
        ])
    def sk(el_hbm, r_hbm, z_hbm, out_hbm, ridx, row, acc):
        cid = lax.axis_index("c")
        sid = lax.axis_index("s")
        wid = cid * _NS + sid
        pltpu.sync_copy(z_hbm.at[pl.ds(sid * rpt, rpt)],
                        acc.at[pl.ds(sid * rpt, rpt)])
        if rem:
            @pl.when(sid == 0)
            def _():
                pltpu.sync_copy(z_hbm.at[pl.ds(rpt * _NS, rem)],
                                acc.at[pl.ds(rpt * _NS, rem)])
        plsc.subcore_barrier()

        def body(j, carry):
            c = wid + j * _NW

            @pl.when(c < nchunks)
            def _():
                base = c * _CH
                pltpu.sync_copy(r_hbm.at[pl.ds(base, _CH)], ridx)
                pltpu.sync_copy(el_hbm.at[pl.ds(base, _CH)], row)
                pltpu.sync_copy(row, acc.at[ridx], add=True)
            return carry

        lax.fori_loop(0, maxiter, body, 0)
        plsc.subcore_barrier()
        pltpu.sync_copy(acc.at[pl.ds(sid * rpt, rpt)],
                        out_hbm.at[cid, pl.ds(sid * rpt, rpt)])
        if rem:
            @pl.when(sid == 0)
            def _():
                pltpu.sync_copy(acc.at[pl.ds(rpt * _NS, rem)],
                                out_hbm.at[cid, pl.ds(rpt * _NS, rem)])

    out = sk(el, receivers, jnp.zeros((n, d), F32))
    return out[0], out[1]


def _scatter_partials_jnp(el, receivers, n):
    agg = jax.ops.segment_sum(el, receivers, num_segments=n)
    return agg, jnp.zeros_like(agg)


def _normalize_feat(x, eps=1e-8):
    mean = jnp.mean(x, axis=0, keepdims=True)
    second = jnp.mean(x * x, axis=0, keepdims=True)
    std = jnp.sqrt(jnp.maximum(second - mean * mean, 0.0))
    return (x - mean) / jnp.maximum(std, eps)


def _safe_norm2(x):
    s = jnp.sum(x * x, axis=-1, keepdims=True)
    out = jnp.sqrt(jnp.where(s > 0, s, 1.0))
    return jnp.where(s > 0, out, 0.0)


def kernel(node_type, pressure, target_pressure, mesh_pos, world_pos, senders,
           receivers, is_trainning, params):
    n = node_type.shape[0]
    e = senders.shape[0]
    steps = len(params["blocks"])

    oh = jax.nn.one_hot(node_type[:, 0], 9, dtype=F32)
    node_features = jnp.concatenate(
        [jnp.zeros((n, 1), F32), _normalize_feat(oh)], axis=-1)
    rel_w = jnp.take(world_pos, senders, axis=0) - jnp.take(world_pos, receivers, axis=0)
    rel_m = jnp.take(mesh_pos, senders, axis=0) - jnp.take(mesh_pos, receivers, axis=0)
    edge_features = _normalize_feat(
        jnp.concatenate([rel_w, _safe_norm2(rel_w), rel_m, _safe_norm2(rel_m)], axis=-1))

    blocks = params["blocks"]
    ew = []
    for t in range(steps):
        em = blocks[t]["edge_mlp"]
        w1 = em["Ws"][0]
        ew.append({
            "w1a": w1[:128], "w1b": w1[128:256], "w1c": w1[256:384],
            "b1": em["bs"][0], "w2": em["Ws"][1], "w3": em["Ws"][2],
            "aux": _pad8([em["bs"][1], em["bs"][2], em["ln_s"], em["ln_b"]]),
        })
        ew[t]["wpq"] = jnp.concatenate([ew[t]["w1a"], ew[t]["w1b"]], axis=1)

    nw = []
    for t in range(steps):
        nm = blocks[t]["node_mlp"]
        nxt = ew[t + 1] if t + 1 < steps else None
        nw.append({
            "v1": nm["Ws"][0], "v2": nm["Ws"][1], "v3": nm["Ws"][2],
            "aux": _pad8([nm["bs"][0], nm["bs"][1], nm["bs"][2], nm["ln_s"], nm["ln_b"]]
                         + ([nxt["b1"]] if nxt is not None else [])),
        })

    ne_grid = n // BN
    ee_grid = e // BE

    enc = params["node_enc"]
    enc_aux = _pad8([enc["bs"][0], enc["bs"][1], enc["bs"][2], enc["ln_s"],
                     enc["ln_b"], ew[0]["b1"]])
    nf = node_features.shape[1]
    nl, p_tab, q_tab = _call_tc(
        _encoder_body, ne_grid,
        [node_features, enc["Ws"][0], enc["Ws"][1], enc["Ws"][2], enc_aux, ew[0]["wpq"]],
        [_rows(BN, nf), _full((nf, 128)), _full((128, 128)), _full((128, 128)),
         _full((8, 128)), _full((128, 256))],
        (jax.ShapeDtypeStruct((n, 128), F32),) * 3,
        (_rows(BN),) * 3,
    )

    eenc = params["edge_enc"]
    eenc_aux = _pad8([eenc["bs"][0], eenc["bs"][1], eenc["bs"][2], eenc["ln_s"], eenc["ln_b"]])
    ef = edge_features.shape[1]
    el = _call_tc(
        _edge_enc_body, ee_grid,
        [edge_features, eenc["Ws"][0], eenc["Ws"][1], eenc["Ws"][2], eenc_aux],
        [_rows(BE, ef), _full((ef, 128)), _full((128, 128)), _full((128, 128)), _full((8, 128))],
        jax.ShapeDtypeStruct((e, 128), F32),
        _rows(BE),
    )

    for t in range(steps):
        gs, gr = _gather_pq(p_tab, q_tab, senders, receivers)
        el = _call_tc(
            _edge_step_body, ee_grid,
            [gs, gr, el, ew[t]["w1c"], ew[t]["w2"], ew[t]["w3"], ew[t]["aux"]],
            [_rows(BE)] * 3 + [_full((128, 128))] * 3 + [_full((8, 128))],
            jax.ShapeDtypeStruct((e, 128), F32),
            _rows(BE),
        )
        a0, a1 = _scatter_partials(el, receivers, n)
        if t + 1 < steps:
            nl, p_tab, q_tab = _call_tc(
                _node_step_body, ne_grid,
                [nl, a0, a1, nw[t]["v1"], nw[t]["v2"], nw[t]["v3"], nw[t]["aux"],
                 ew[t + 1]["wpq"]],
                [_rows(BN)] * 3 + [_full((256, 128)), _full((128, 128)), _full((128, 128)),
                                   _full((8, 128)), _full((128, 256))],
                (jax.ShapeDtypeStruct((n, 128), F32),) * 3,
                (_rows(BN),) * 3,
            )
        else:
            dec = params["decoder"]
            daux = _pad8([dec["bs"][0], dec["bs"][1],
                          jnp.pad(dec["bs"][2], (0, 125))])
            out = _call_tc(
                _node_last_body, ne_grid,
                [nl, a0, a1, nw[t]["v1"], nw[t]["v2"], nw[t]["v3"], nw[t]["aux"],
                 dec["Ws"][0], dec["Ws"][1], dec["Ws"][2], daux],
                [_rows(BN)] * 3 + [_full((256, 128)), _full((128, 128)), _full((128, 128)),
                                   _full((8, 128)), _full((128, 128)), _full((128, 128)),
                                   _full((128, 3)), _full((8, 128))],
                jax.ShapeDtypeStruct((n, 3), F32),
                _rows(BN, 3),
            )
    return out


# speedup vs baseline: 3.6349x; 1.2092x over previous
import functools

import jax
import jax.numpy as jnp
from jax import lax
from jax.experimental import pallas as pl
from jax.experimental.pallas import tpu as pltpu
from jax.experimental.pallas import tpu_sc as plsc

F32 = jnp.float32

BE = 2000
BN = 2000


def _ln(x, g, b):
    mu = jnp.mean(x, axis=1, keepdims=True)
    var = jnp.mean((x - mu) * (x - mu), axis=1, keepdims=True)
    return (x - mu) / jnp.sqrt(var + 1e-5) * g + b


def _relu(x):
    return jnp.maximum(x, 0.0)


def _dot(x, w):
    return jnp.dot(x, w, preferred_element_type=F32)


def _full(shape):
    return pl.BlockSpec(shape, lambda i: (0,) * len(shape))


def _rows(block, minor=128):
    return pl.BlockSpec((block, minor), lambda i: (i, 0))


def _encoder_body(f_ref, w1, w2, w3, aux, wpq, out, p_out, q_out):
    a = aux[...]
    x = _relu(_dot(f_ref[...], w1[...]) + a[0:1])
    x = _relu(_dot(x, w2[...]) + a[1:2])
    x = _dot(x, w3[...]) + a[2:3]
    x = _ln(x, a[3:4], a[4:5])
    out[...] = x
    pq = _dot(x, wpq[...])
    p_out[...] = pq[:, :128] + a[5:6]
    q_out[...] = pq[:, 128:]


def _edge_enc_body(f_ref, w1, w2, w3, aux, out):
    a = aux[...]
    x = _relu(_dot(f_ref[...], w1[...]) + a[0:1])
    x = _relu(_dot(x, w2[...]) + a[1:2])
    x = _dot(x, w3[...]) + a[2:3]
    out[...] = _ln(x, a[3:4], a[4:5])


def _edge_step_body(gs, gr, el, w1c, w2, w3, aux, out):
    a = aux[...]
    elv = el[...]
    x = _relu(gs[...] + gr[...] + _dot(elv, w1c[...]))
    x = _relu(_dot(x, w2[...]) + a[0:1])
    x = _dot(x, w3[...]) + a[1:2]
    out[...] = _ln(x, a[2:3], a[3:4]) + elv


def _node_step_body(nl, a0, a1, v1, v2, v3, aux, wpq, out, p_out, q_out):
    a = aux[...]
    nlv = nl[...]
    x = jnp.concatenate([nlv, a0[...] + a1[...]], axis=1)
    x = _relu(_dot(x, v1[...]) + a[0:1])
    x = _relu(_dot(x, v2[...]) + a[1:2])
    x = _dot(x, v3[...]) + a[2:3]
    x = _ln(x, a[3:4], a[4:5]) + nlv
    out[...] = x
    pq = _dot(x, wpq[...])
    p_out[...] = pq[:, :128] + a[5:6]
    q_out[...] = pq[:, 128:]


def _node_last_body(nl, a0, a1, v1, v2, v3, aux, d1, d2, d3, daux, out):
    a = aux[...]
    da = daux[...]
    nlv = nl[...]
    x = jnp.concatenate([nlv, a0[...] + a1[...]], axis=1)
    x = _relu(_dot(x, v1[...]) + a[0:1])
    x = _relu(_dot(x, v2[...]) + a[1:2])
    x = _dot(x, v3[...]) + a[2:3]
    x = _ln(x, a[3:4], a[4:5]) + nlv
    y = _relu(_dot(x, d1[...]) + da[0:1])
    y = _relu(_dot(y, d2[...]) + da[1:2])
    y = _dot(y, d3[...]) + da[2:3, :3]
    out[...] = y


def _pad8(rows):
    out = jnp.zeros((8, 128), F32)
    for i, r in enumerate(rows):
        out = out.at[i, : r.shape[0]].set(r)
    return out


def _call_tc(body, grid, in_arrays, in_specs, out_shapes, out_specs):
    return pl.pallas_call(
        body,
        grid=(grid,),
        in_specs=in_specs,
        out_specs=out_specs,
        out_shape=out_shapes,
        compiler_params=pltpu.CompilerParams(
            dimension_semantics=("arbitrary",),
        ),
    )(*in_arrays)


_NC = 2
_NS = 16
_NW = _NC * _NS
_CH = 128


def _gather_pq(p_tab, q_tab, senders, receivers):
    e, d = p_tab.shape[0], p_tab.shape[1]
    e_edges = senders.shape[0]
    nchunks = e_edges // _CH
    maxiter = (nchunks + _NW - 1) // _NW
    mesh = plsc.VectorSubcoreMesh(core_axis_name="c", subcore_axis_name="s")

    assert maxiter % 2 == 0

    @functools.partial(
        pl.kernel, mesh=mesh,
        out_type=(jax.ShapeDtypeStruct((e_edges, d), p_tab.dtype),
                  jax.ShapeDtypeStruct((e_edges, d), q_tab.dtype)),
        scratch_types=[
            pltpu.VMEM((2, _CH), jnp.int32), pltpu.VMEM((2, _CH), jnp.int32),
            pltpu.VMEM((2, _CH, d), p_tab.dtype), pltpu.VMEM((2, _CH, d), q_tab.dtype),
            pltpu.SemaphoreType.DMA, pltpu.SemaphoreType.DMA,
            pltpu.SemaphoreType.DMA, pltpu.SemaphoreType.DMA,
            pltpu.SemaphoreType.DMA, pltpu.SemaphoreType.DMA,
        ])
    def gk(p_hbm, q_hbm, s_hbm, r_hbm, op_hbm, oq_hbm, sidx, ridx, prow, qrow,
           semi0, semi1, semg0, semg1, semw0, semw1):
        wid = lax.axis_index("c") * _NS + lax.axis_index("s")
        semi = (semi0, semi1)
        semg = (semg0, semg1)
        semw = (semw0, semw1)

        def c_of(j):
            return wid + j * _NW

        def idx_copies(j, s):
            base = c_of(j) * _CH
            return (pltpu.make_async_copy(s_hbm.at[pl.ds(base, _CH)], sidx.at[s], semi[s]),
                    pltpu.make_async_copy(r_hbm.at[pl.ds(base, _CH)], ridx.at[s], semi[s]))

        def gather_copies(j, s):
            del j
            return (pltpu.make_async_copy(p_hbm.at[sidx.at[s]], prow.at[s], semg[s]),
                    pltpu.make_async_copy(q_hbm.at[ridx.at[s]], qrow.at[s], semg[s]))

        def wb_copies(j, s):
            base = c_of(j) * _CH
            return (pltpu.make_async_copy(prow.at[s], op_hbm.at[pl.ds(base, _CH)], semw[s]),
                    pltpu.make_async_copy(qrow.at[s], oq_hbm.at[pl.ds(base, _CH)], semw[s]))

        def start(mk, j, s):
            for cp in mk(j, s):
                cp.start()

        def wait(mk, j, s):
            for cp in mk(j, s):
                cp.wait()

        def guarded(cond, fn, mk, j, s):
            @pl.when(cond)
            def _():
                fn(mk, j, s)

        start(idx_copies, 0, 0)

        def body(k, carry):
            for s in (0, 1):
                j = 2 * k + s
                c = c_of(j)
                not_first = k >= 1 if s == 0 else True
                guarded(jnp.logical_and(k >= 1, c - 2 * _NW < nchunks),
                        wait, wb_copies, j - 2, s)
                guarded(c < nchunks, wait, idx_copies, j, s)
                guarded(c < nchunks, start, gather_copies, j, s)
                guarded(jnp.logical_and(not_first, c - _NW < nchunks),
                        wait, gather_copies, j - 1, 1 - s)
                guarded(jnp.logical_and(not_first, c - _NW < nchunks),
                        start, wb_copies, j - 1, 1 - s)
                guarded(c + _NW < nchunks, start, idx_copies, j + 1, 1 - s)
            return carry

        lax.fori_loop(0, maxiter // 2, body, 0)
        jl = maxiter - 1
        guarded(c_of(jl) < nchunks, wait, gather_copies, jl, jl % 2)
        guarded(c_of(jl) < nchunks, start, wb_copies, jl, jl % 2)
        guarded(c_of(jl - 1) < nchunks, wait, wb_copies, jl - 1, (jl - 1) % 2)
        guarded(c_of(jl) < nchunks, wait, wb_copies, jl, jl % 2)

    return gk(p_tab, q_tab, senders, receivers)


def _scatter_partials(el, receivers, n):
    e_edges, d = el.shape
    nchunks = e_edges // _CH
    maxiter = (nchunks + _NW - 1) // _NW
    rpt = (n // (8 * _NS)) * 8
    rem = n - rpt * _NS
    mesh = plsc.VectorSubcoreMesh(core_axis_name="c", subcore_axis_name="s")

    assert maxiter % 2 == 0

    @functools.partial(
        pl.kernel, mesh=mesh,
        out_type=jax.ShapeDtypeStruct((_NC, n, d), F32),
        scratch_types=[
            pltpu.VMEM((2, _CH), jnp.int32),
            pltpu.VMEM((2, _CH, d), F32),
            pltpu.VMEM_SHARED((n, d), F32),
            pltpu.SemaphoreType.DMA, pltpu.SemaphoreType.DMA,
            pltpu.SemaphoreType.DMA, pltpu.SemaphoreType.DMA,
            pltpu.SemaphoreType.DMA,
        ])
    def sk(el_hbm, r_hbm, z_hbm, out_hbm, ridx, row, acc,
           seml0, seml1, sems0, sems1, semz):
        cid = lax.axis_index("c")
        sid = lax.axis_index("s")
        wid = cid * _NS + sid
        seml = (seml0, seml1)
        sems = (sems0, sems1)

        def c_of(j):
            return wid + j * _NW

        def load_copies(j, s):
            base = c_of(j) * _CH
            return (pltpu.make_async_copy(r_hbm.at[pl.ds(base, _CH)], ridx.at[s], seml[s]),
                    pltpu.make_async_copy(el_hbm.at[pl.ds(base, _CH)], row.at[s], seml[s]))

        def scat_start(s):
            pltpu.async_copy(row.at[s], acc.at[ridx.at[s]], sems[s], add=True)

        def scat_wait(s):
            pltpu.make_async_copy(row.at[s], acc.at[ridx.at[s]], sems[s]).wait()

        def start(mk, j, s):
            for cp in mk(j, s):
                cp.start()

        def wait(mk, j, s):
            for cp in mk(j, s):
                cp.wait()

        def guarded(cond, fn, *args):
            @pl.when(cond)
            def _():
                fn(*args)

        start(load_copies, 0, 0)
        zinit = pltpu.make_async_copy(z_hbm.at[pl.ds(sid * rpt, rpt)],
                                      acc.at[pl.ds(sid * rpt, rpt)], semz)
        zinit.start()
        if rem:
            zrem = pltpu.make_async_copy(z_hbm.at[pl.ds(rpt * _NS, rem)],
                                         acc.at[pl.ds(rpt * _NS, rem)], semz)

            @pl.when(sid == 0)
            def _():
                zrem.start()
                zrem.wait()
        zinit.wait()
        plsc.subcore_barrier()

        def body(k, carry):
            for s in (0, 1):
                j = 2 * k + s
                c = c_of(j)
                not_first = k >= 1 if s == 0 else True
                guarded(c < nchunks, wait, load_copies, j, s)
                guarded(c < nchunks, scat_start, s)
                guarded(jnp.logical_and(not_first, c - _NW < nchunks),
                        scat_wait, 1 - s)
                guarded(c + _NW < nchunks, start, load_copies, j + 1, 1 - s)
            return carry

        lax.fori_loop(0, maxiter // 2, body, 0)
        jl = maxiter - 1
        guarded(c_of(jl) < nchunks, scat_wait, jl % 2)
        plsc.subcore_barrier()
        pltpu.sync_copy(acc.at[pl.ds(sid * rpt, rpt)],
                        out_hbm.at[cid, pl.ds(sid * rpt, rpt)])
        if rem:
            @pl.when(sid == 0)
            def _():
                pltpu.sync_copy(acc.at[pl.ds(rpt * _NS, rem)],
                                out_hbm.at[cid, pl.ds(rpt * _NS, rem)])

    out = sk(el, receivers, jnp.zeros((n, d), F32))
    return out[0], out[1]


def _scatter_partials_jnp(el, receivers, n):
    agg = jax.ops.segment_sum(el, receivers, num_segments=n)
    return agg, jnp.zeros_like(agg)


def _normalize_feat(x, eps=1e-8):
    mean = jnp.mean(x, axis=0, keepdims=True)
    second = jnp.mean(x * x, axis=0, keepdims=True)
    std = jnp.sqrt(jnp.maximum(second - mean * mean, 0.0))
    return (x - mean) / jnp.maximum(std, eps)


def _safe_norm2(x):
    s = jnp.sum(x * x, axis=-1, keepdims=True)
    out = jnp.sqrt(jnp.where(s > 0, s, 1.0))
    return jnp.where(s > 0, out, 0.0)


def kernel(node_type, pressure, target_pressure, mesh_pos, world_pos, senders,
           receivers, is_trainning, params):
    n = node_type.shape[0]
    e = senders.shape[0]
    steps = len(params["blocks"])

    oh = jax.nn.one_hot(node_type[:, 0], 9, dtype=F32)
    node_features = jnp.concatenate(
        [jnp.zeros((n, 1), F32), _normalize_feat(oh)], axis=-1)
    rel_w = jnp.take(world_pos, senders, axis=0) - jnp.take(world_pos, receivers, axis=0)
    rel_m = jnp.take(mesh_pos, senders, axis=0) - jnp.take(mesh_pos, receivers, axis=0)
    edge_features = _normalize_feat(
        jnp.concatenate([rel_w, _safe_norm2(rel_w), rel_m, _safe_norm2(rel_m)], axis=-1))

    blocks = params["blocks"]
    ew = []
    for t in range(steps):
        em = blocks[t]["edge_mlp"]
        w1 = em["Ws"][0]
        ew.append({
            "w1a": w1[:128], "w1b": w1[128:256], "w1c": w1[256:384],
            "b1": em["bs"][0], "w2": em["Ws"][1], "w3": em["Ws"][2],
            "aux": _pad8([em["bs"][1], em["bs"][2], em["ln_s"], em["ln_b"]]),
        })
        ew[t]["wpq"] = jnp.concatenate([ew[t]["w1a"], ew[t]["w1b"]], axis=1)

    nw = []
    for t in range(steps):
        nm = blocks[t]["node_mlp"]
        nxt = ew[t + 1] if t + 1 < steps else None
        nw.append({
            "v1": nm["Ws"][0], "v2": nm["Ws"][1], "v3": nm["Ws"][2],
            "aux": _pad8([nm["bs"][0], nm["bs"][1], nm["bs"][2], nm["ln_s"], nm["ln_b"]]
                         + ([nxt["b1"]] if nxt is not None else [])),
        })

    ne_grid = n // BN
    ee_grid = e // BE

    enc = params["node_enc"]
    enc_aux = _pad8([enc["bs"][0], enc["bs"][1], enc["bs"][2], enc["ln_s"],
                     enc["ln_b"], ew[0]["b1"]])
    nf = node_features.shape[1]
    nl, p_tab, q_tab = _call_tc(
        _encoder_body, ne_grid,
        [node_features, enc["Ws"][0], enc["Ws"][1], enc["Ws"][2], enc_aux, ew[0]["wpq"]],
        [_rows(BN, nf), _full((nf, 128)), _full((128, 128)), _full((128, 128)),
         _full((8, 128)), _full((128, 256))],
        (jax.ShapeDtypeStruct((n, 128), F32),) * 3,
        (_rows(BN),) * 3,
    )

    eenc = params["edge_enc"]
    eenc_aux = _pad8([eenc["bs"][0], eenc["bs"][1], eenc["bs"][2], eenc["ln_s"], eenc["ln_b"]])
    ef = edge_features.shape[1]
    el = _call_tc(
        _edge_enc_body, ee_grid,
        [edge_features, eenc["Ws"][0], eenc["Ws"][1], eenc["Ws"][2], eenc_aux],
        [_rows(BE, ef), _full((ef, 128)), _full((128, 128)), _full((128, 128)), _full((8, 128))],
        jax.ShapeDtypeStruct((e, 128), F32),
        _rows(BE),
    )

    for t in range(steps):
        gs, gr = _gather_pq(p_tab, q_tab, senders, receivers)
        el = _call_tc(
            _edge_step_body, ee_grid,
            [gs, gr, el, ew[t]["w1c"], ew[t]["w2"], ew[t]["w3"], ew[t]["aux"]],
            [_rows(BE)] * 3 + [_full((128, 128))] * 3 + [_full((8, 128))],
            jax.ShapeDtypeStruct((e, 128), F32),
            _rows(BE),
        )
        a0, a1 = _scatter_partials(el, receivers, n)
        if t + 1 < steps:
            nl, p_tab, q_tab = _call_tc(
                _node_step_body, ne_grid,
                [nl, a0, a1, nw[t]["v1"], nw[t]["v2"], nw[t]["v3"], nw[t]["aux"],
                 ew[t + 1]["wpq"]],
                [_rows(BN)] * 3 + [_full((256, 128)), _full((128, 128)), _full((128, 128)),
                                   _full((8, 128)), _full((128, 256))],
                (jax.ShapeDtypeStruct((n, 128), F32),) * 3,
                (_rows(BN),) * 3,
            )
        else:
            dec = params["decoder"]
            daux = _pad8([dec["bs"][0], dec["bs"][1],
                          jnp.pad(dec["bs"][2], (0, 125))])
            out = _call_tc(
                _node_last_body, ne_grid,
                [nl, a0, a1, nw[t]["v1"], nw[t]["v2"], nw[t]["v3"], nw[t]["aux"],
                 dec["Ws"][0], dec["Ws"][1], dec["Ws"][2], daux],
                [_rows(BN)] * 3 + [_full((256, 128)), _full((128, 128)), _full((128, 128)),
                                   _full((8, 128)), _full((128, 128)), _full((128, 128)),
                                   _full((128, 3)), _full((8, 128))],
                jax.ShapeDtypeStruct((n, 3), F32),
                _rows(BN, 3),
            )
    return out


# speedup vs baseline: 3.6458x; 1.0030x over previous
import functools

import jax
import jax.numpy as jnp
from jax import lax
from jax.experimental import pallas as pl
from jax.experimental.pallas import tpu as pltpu
from jax.experimental.pallas import tpu_sc as plsc

F32 = jnp.float32

BE = 2000
BN = 2000


def _ln(x, g, b):
    mu = jnp.mean(x, axis=1, keepdims=True)
    var = jnp.mean((x - mu) * (x - mu), axis=1, keepdims=True)
    return (x - mu) / jnp.sqrt(var + 1e-5) * g + b


def _relu(x):
    return jnp.maximum(x, 0.0)


def _dot(x, w):
    return jnp.dot(x, w, preferred_element_type=F32)


def _full(shape):
    return pl.BlockSpec(shape, lambda i: (0,) * len(shape))


def _rows(block, minor=128):
    return pl.BlockSpec((block, minor), lambda i: (i, 0))


def _encoder_body(x1_ref, w2, w3, aux, wpq, out, p_out, q_out):
    a = aux[...]
    x = _relu(_dot(x1_ref[...], w2[...]) + a[1:2])
    x = _dot(x, w3[...]) + a[2:3]
    x = _ln(x, a[3:4], a[4:5])
    out[...] = x
    pq = _dot(x, wpq[...])
    p_out[...] = pq[:, :128] + a[5:6]
    q_out[...] = pq[:, 128:]


def _edge_enc_body(f_ref, w1, w2, w3, aux, out):
    a = aux[...]
    x = _relu(_dot(f_ref[...], w1[...]) + a[0:1])
    x = _relu(_dot(x, w2[...]) + a[1:2])
    x = _dot(x, w3[...]) + a[2:3]
    out[...] = _ln(x, a[3:4], a[4:5])


def _edge_step_body(gs, gr, el, w1c, w2, w3, aux, out):
    a = aux[...]
    elv = el[...]
    x = _relu(gs[...] + gr[...] + _dot(elv, w1c[...]))
    x = _relu(_dot(x, w2[...]) + a[0:1])
    x = _dot(x, w3[...]) + a[1:2]
    out[...] = _ln(x, a[2:3], a[3:4]) + elv


def _node_step_body(nl, a0, a1, v1, v2, v3, aux, wpq, out, p_out, q_out):
    a = aux[...]
    nlv = nl[...]
    x = jnp.concatenate([nlv, a0[...] + a1[...]], axis=1)
    x = _relu(_dot(x, v1[...]) + a[0:1])
    x = _relu(_dot(x, v2[...]) + a[1:2])
    x = _dot(x, v3[...]) + a[2:3]
    x = _ln(x, a[3:4], a[4:5]) + nlv
    out[...] = x
    pq = _dot(x, wpq[...])
    p_out[...] = pq[:, :128] + a[5:6]
    q_out[...] = pq[:, 128:]


def _node_last_body(nl, a0, a1, v1, v2, v3, aux, d1, d2, d3, daux, out):
    a = aux[...]
    da = daux[...]
    nlv = nl[...]
    x = jnp.concatenate([nlv, a0[...] + a1[...]], axis=1)
    x = _relu(_dot(x, v1[...]) + a[0:1])
    x = _relu(_dot(x, v2[...]) + a[1:2])
    x = _dot(x, v3[...]) + a[2:3]
    x = _ln(x, a[3:4], a[4:5]) + nlv
    y = _relu(_dot(x, d1[...]) + da[0:1])
    y = _relu(_dot(y, d2[...]) + da[1:2])
    y = _dot(y, d3[...]) + da[2:3, :3]
    out[...] = y


def _pad8(rows):
    out = jnp.zeros((8, 128), F32)
    for i, r in enumerate(rows):
        out = out.at[i, : r.shape[0]].set(r)
    return out


def _call_tc(body, grid, in_arrays, in_specs, out_shapes, out_specs):
    return pl.pallas_call(
        body,
        grid=(grid,),
        in_specs=in_specs,
        out_specs=out_specs,
        out_shape=out_shapes,
        compiler_params=pltpu.CompilerParams(
            dimension_semantics=("arbitrary",),
        ),
    )(*in_arrays)


_NC = 2
_NS = 16
_NW = _NC * _NS
_CH = 128


def _gather_pq(p_tab, q_tab, senders, receivers):
    e, d = p_tab.shape[0], p_tab.shape[1]
    e_edges = senders.shape[0]
    nchunks = e_edges // _CH
    maxiter = (nchunks + _NW - 1) // _NW
    mesh = plsc.VectorSubcoreMesh(core_axis_name="c", subcore_axis_name="s")

    assert maxiter % 2 == 0

    @functools.partial(
        pl.kernel, mesh=mesh,
        out_type=(jax.ShapeDtypeStruct((e_edges, d), p_tab.dtype),
                  jax.ShapeDtypeStruct((e_edges, d), q_tab.dtype)),
        scratch_types=[
            pltpu.VMEM((2, _CH), jnp.int32), pltpu.VMEM((2, _CH), jnp.int32),
            pltpu.VMEM((2, _CH, d), p_tab.dtype), pltpu.VMEM((2, _CH, d), q_tab.dtype),
            pltpu.SemaphoreType.DMA, pltpu.SemaphoreType.DMA,
            pltpu.SemaphoreType.DMA, pltpu.SemaphoreType.DMA,
            pltpu.SemaphoreType.DMA, pltpu.SemaphoreType.DMA,
        ])
    def gk(p_hbm, q_hbm, s_hbm, r_hbm, op_hbm, oq_hbm, sidx, ridx, prow, qrow,
           semi0, semi1, semg0, semg1, semw0, semw1):
        wid = lax.axis_index("c") * _NS + lax.axis_index("s")
        semi = (semi0, semi1)
        semg = (semg0, semg1)
        semw = (semw0, semw1)

        def c_of(j):
            return wid + j * _NW

        def idx_copies(j, s):
            base = c_of(j) * _CH
            return (pltpu.make_async_copy(s_hbm.at[pl.ds(base, _CH)], sidx.at[s], semi[s]),
                    pltpu.make_async_copy(r_hbm.at[pl.ds(base, _CH)], ridx.at[s], semi[s]))

        def gather_copies(j, s):
            del j
            return (pltpu.make_async_copy(p_hbm.at[sidx.at[s]], prow.at[s], semg[s]),
                    pltpu.make_async_copy(q_hbm.at[ridx.at[s]], qrow.at[s], semg[s]))

        def wb_copies(j, s):
            base = c_of(j) * _CH
            return (pltpu.make_async_copy(prow.at[s], op_hbm.at[pl.ds(base, _CH)], semw[s]),
                    pltpu.make_async_copy(qrow.at[s], oq_hbm.at[pl.ds(base, _CH)], semw[s]))

        def start(mk, j, s):
            for cp in mk(j, s):
                cp.start()

        def wait(mk, j, s):
            for cp in mk(j, s):
                cp.wait()

        def guarded(cond, fn, mk, j, s):
            @pl.when(cond)
            def _():
                fn(mk, j, s)

        start(idx_copies, 0, 0)

        def body(k, carry):
            for s in (0, 1):
                j = 2 * k + s
                c = c_of(j)
                not_first = k >= 1 if s == 0 else True
                guarded(jnp.logical_and(k >= 1, c - 2 * _NW < nchunks),
                        wait, wb_copies, j - 2, s)
                guarded(c < nchunks, wait, idx_copies, j, s)
                guarded(c < nchunks, start, gather_copies, j, s)
                guarded(jnp.logical_and(not_first, c - _NW < nchunks),
                        wait, gather_copies, j - 1, 1 - s)
                guarded(jnp.logical_and(not_first, c - _NW < nchunks),
                        start, wb_copies, j - 1, 1 - s)
                guarded(c + _NW < nchunks, start, idx_copies, j + 1, 1 - s)
            return carry

        lax.fori_loop(0, maxiter // 2, body, 0)
        jl = maxiter - 1
        guarded(c_of(jl) < nchunks, wait, gather_copies, jl, jl % 2)
        guarded(c_of(jl) < nchunks, start, wb_copies, jl, jl % 2)
        guarded(c_of(jl - 1) < nchunks, wait, wb_copies, jl - 1, (jl - 1) % 2)
        guarded(c_of(jl) < nchunks, wait, wb_copies, jl, jl % 2)

    return gk(p_tab, q_tab, senders, receivers)


def _scatter_partials(el, receivers, n):
    e_edges, d = el.shape
    nchunks = e_edges // _CH
    maxiter = (nchunks + _NW - 1) // _NW
    rpt = (n // (8 * _NS)) * 8
    rem = n - rpt * _NS
    mesh = plsc.VectorSubcoreMesh(core_axis_name="c", subcore_axis_name="s")

    assert maxiter % 2 == 0

    @functools.partial(
        pl.kernel, mesh=mesh,
        out_type=jax.ShapeDtypeStruct((_NC, n, d), F32),
        scratch_types=[
            pltpu.VMEM((2, _CH), jnp.int32),
            pltpu.VMEM((2, _CH, d), F32),
            pltpu.VMEM_SHARED((n, d), F32),
            pltpu.SemaphoreType.DMA, pltpu.SemaphoreType.DMA,
            pltpu.SemaphoreType.DMA, pltpu.SemaphoreType.DMA,
            pltpu.SemaphoreType.DMA,
        ])
    def sk(el_hbm, r_hbm, z_hbm, out_hbm, ridx, row, acc,
           seml0, seml1, sems0, sems1, semz):
        cid = lax.axis_index("c")
        sid = lax.axis_index("s")
        wid = cid * _NS + sid
        seml = (seml0, seml1)
        sems = (sems0, sems1)

        def c_of(j):
            return wid + j * _NW

        def load_copies(j, s):
            base = c_of(j) * _CH
            return (pltpu.make_async_copy(r_hbm.at[pl.ds(base, _CH)], ridx.at[s], seml[s]),
                    pltpu.make_async_copy(el_hbm.at[pl.ds(base, _CH)], row.at[s], seml[s]))

        def scat_start(s):
            pltpu.async_copy(row.at[s], acc.at[ridx.at[s]], sems[s], add=True)

        def scat_wait(s):
            pltpu.make_async_copy(row.at[s], acc.at[ridx.at[s]], sems[s]).wait()

        def start(mk, j, s):
            for cp in mk(j, s):
                cp.start()

        def wait(mk, j, s):
            for cp in mk(j, s):
                cp.wait()

        def guarded(cond, fn, *args):
            @pl.when(cond)
            def _():
                fn(*args)

        start(load_copies, 0, 0)
        zinit = pltpu.make_async_copy(z_hbm.at[pl.ds(sid * rpt, rpt)],
                                      acc.at[pl.ds(sid * rpt, rpt)], semz)
        zinit.start()
        if rem:
            zrem = pltpu.make_async_copy(z_hbm.at[pl.ds(rpt * _NS, rem)],
                                         acc.at[pl.ds(rpt * _NS, rem)], semz)

            @pl.when(sid == 0)
            def _():
                zrem.start()
                zrem.wait()
        zinit.wait()
        plsc.subcore_barrier()

        def body(k, carry):
            for s in (0, 1):
                j = 2 * k + s
                c = c_of(j)
                not_first = k >= 1 if s == 0 else True
                guarded(c < nchunks, wait, load_copies, j, s)
                guarded(c < nchunks, scat_start, s)
                guarded(jnp.logical_and(not_first, c - _NW < nchunks),
                        scat_wait, 1 - s)
                guarded(c + _NW < nchunks, start, load_copies, j + 1, 1 - s)
            return carry

        lax.fori_loop(0, maxiter // 2, body, 0)
        jl = maxiter - 1
        guarded(c_of(jl) < nchunks, scat_wait, jl % 2)
        plsc.subcore_barrier()
        pltpu.sync_copy(acc.at[pl.ds(sid * rpt, rpt)],
                        out_hbm.at[cid, pl.ds(sid * rpt, rpt)])
        if rem:
            @pl.when(sid == 0)
            def _():
                pltpu.sync_copy(acc.at[pl.ds(rpt * _NS, rem)],
                                out_hbm.at[cid, pl.ds(rpt * _NS, rem)])

    out = sk(el, receivers, jnp.zeros((n, d), F32))
    return out[0], out[1]


def _scatter_partials_jnp(el, receivers, n):
    agg = jax.ops.segment_sum(el, receivers, num_segments=n)
    return agg, jnp.zeros_like(agg)


def _normalize_feat(x, eps=1e-8):
    mean = jnp.mean(x, axis=0, keepdims=True)
    second = jnp.mean(x * x, axis=0, keepdims=True)
    std = jnp.sqrt(jnp.maximum(second - mean * mean, 0.0))
    return (x - mean) / jnp.maximum(std, eps)


def _safe_norm2(x):
    s = jnp.sum(x * x, axis=-1, keepdims=True)
    out = jnp.sqrt(jnp.where(s > 0, s, 1.0))
    return jnp.where(s > 0, out, 0.0)


def kernel(node_type, pressure, target_pressure, mesh_pos, world_pos, senders,
           receivers, is_trainning, params):
    n = node_type.shape[0]
    e = senders.shape[0]
    steps = len(params["blocks"])

    pressure_increase = target_pressure - pressure
    expanded = jnp.broadcast_to(pressure_increase, (n, 1))
    oh = jax.nn.one_hot(node_type[:, 0], 9, dtype=F32)
    node_features = _normalize_feat(jnp.concatenate([expanded, oh], axis=-1))
    rel_w = jnp.take(world_pos, senders, axis=0) - jnp.take(world_pos, receivers, axis=0)
    rel_m = jnp.take(mesh_pos, senders, axis=0) - jnp.take(mesh_pos, receivers, axis=0)
    edge_features = _normalize_feat(
        jnp.concatenate([rel_w, _safe_norm2(rel_w), rel_m, _safe_norm2(rel_m)], axis=-1))

    blocks = params["blocks"]
    ew = []
    for t in range(steps):
        em = blocks[t]["edge_mlp"]
        w1 = em["Ws"][0]
        ew.append({
            "w1a": w1[:128], "w1b": w1[128:256], "w1c": w1[256:384],
            "b1": em["bs"][0], "w2": em["Ws"][1], "w3": em["Ws"][2],
            "aux": _pad8([em["bs"][1], em["bs"][2], em["ln_s"], em["ln_b"]]),
        })
        ew[t]["wpq"] = jnp.concatenate([ew[t]["w1a"], ew[t]["w1b"]], axis=1)

    nw = []
    for t in range(steps):
        nm = blocks[t]["node_mlp"]
        nxt = ew[t + 1] if t + 1 < steps else None
        nw.append({
            "v1": nm["Ws"][0], "v2": nm["Ws"][1], "v3": nm["Ws"][2],
            "aux": _pad8([nm["bs"][0], nm["bs"][1], nm["bs"][2], nm["ln_s"], nm["ln_b"]]
                         + ([nxt["b1"]] if nxt is not None else [])),
        })

    ne_grid = n // BN
    ee_grid = e // BE

    enc = params["node_enc"]
    enc_aux = _pad8([enc["bs"][0], enc["bs"][1], enc["bs"][2], enc["ln_s"],
                     enc["ln_b"], ew[0]["b1"]])
    x1n = jax.nn.relu(node_features @ enc["Ws"][0] + enc["bs"][0])
    nl, p_tab, q_tab = _call_tc(
        _encoder_body, ne_grid,
        [x1n, enc["Ws"][1], enc["Ws"][2], enc_aux, ew[0]["wpq"]],
        [_rows(BN), _full((128, 128)), _full((128, 128)),
         _full((8, 128)), _full((128, 256))],
        (jax.ShapeDtypeStruct((n, 128), F32),) * 3,
        (_rows(BN),) * 3,
    )

    eenc = params["edge_enc"]
    eenc_aux = _pad8([eenc["bs"][0], eenc["bs"][1], eenc["bs"][2], eenc["ln_s"], eenc["ln_b"]])
    ef = edge_features.shape[1]
    el = _call_tc(
        _edge_enc_body, ee_grid,
        [edge_features, eenc["Ws"][0], eenc["Ws"][1], eenc["Ws"][2], eenc_aux],
        [_rows(BE, ef), _full((ef, 128)), _full((128, 128)), _full((128, 128)), _full((8, 128))],
        jax.ShapeDtypeStruct((e, 128), F32),
        _rows(BE),
    )

    for t in range(steps):
        gs, gr = _gather_pq(p_tab, q_tab, senders, receivers)
        el = _call_tc(
            _edge_step_body, ee_grid,
            [gs, gr, el, ew[t]["w1c"], ew[t]["w2"], ew[t]["w3"], ew[t]["aux"]],
            [_rows(BE)] * 3 + [_full((128, 128))] * 3 + [_full((8, 128))],
            jax.ShapeDtypeStruct((e, 128), F32),
            _rows(BE),
        )
        a0, a1 = _scatter_partials(el, receivers, n)
        if t + 1 < steps:
            nl, p_tab, q_tab = _call_tc(
                _node_step_body, ne_grid,
                [nl, a0, a1, nw[t]["v1"], nw[t]["v2"], nw[t]["v3"], nw[t]["aux"],
                 ew[t + 1]["wpq"]],
                [_rows(BN)] * 3 + [_full((256, 128)), _full((128, 128)), _full((128, 128)),
                                   _full((8, 128)), _full((128, 256))],
                (jax.ShapeDtypeStruct((n, 128), F32),) * 3,
                (_rows(BN),) * 3,
            )
        else:
            dec = params["decoder"]
            daux = _pad8([dec["bs"][0], dec["bs"][1],
                          jnp.pad(dec["bs"][2], (0, 125))])
            out = _call_tc(
                _node_last_body, ne_grid,
                [nl, a0, a1, nw[t]["v1"], nw[t]["v2"], nw[t]["v3"], nw[t]["aux"],
                 dec["Ws"][0], dec["Ws"][1], dec["Ws"][2], daux],
                [_rows(BN)] * 3 + [_full((256, 128)), _full((128, 128)), _full((128, 128)),
                                   _full((8, 128)), _full((128, 128)), _full((128, 128)),
                                   _full((128, 3)), _full((8, 128))],
                jax.ShapeDtypeStruct((n, 3), F32),
                _rows(BN, 3),
            )
    return out


# speedup vs baseline: 3.6483x; 1.0007x over previous
import functools

import jax
import jax.numpy as jnp
from jax import lax
from jax.experimental import pallas as pl
from jax.experimental.pallas import tpu as pltpu
from jax.experimental.pallas import tpu_sc as plsc

F32 = jnp.float32

BE = 2000
BN = 2000


def _ln(x, g, b):
    mu = jnp.mean(x, axis=1, keepdims=True)
    var = jnp.mean((x - mu) * (x - mu), axis=1, keepdims=True)
    return (x - mu) / jnp.sqrt(var + 1e-5) * g + b


def _relu(x):
    return jnp.maximum(x, 0.0)


def _dot(x, w):
    return jnp.dot(x, w, preferred_element_type=F32)


def _full(shape):
    return pl.BlockSpec(shape, lambda i: (0,) * len(shape))


def _rows(block, minor=128):
    return pl.BlockSpec((block, minor), lambda i: (i, 0))


def _encoder_body(x1_ref, w2, w3, aux, wpq, out, p_out, q_out):
    a = aux[...]
    x = _relu(_dot(x1_ref[...], w2[...]) + a[1:2])
    x = _dot(x, w3[...]) + a[2:3]
    x = _ln(x, a[3:4], a[4:5])
    out[...] = x
    pq = _dot(x, wpq[...])
    p_out[...] = pq[:, :128] + a[5:6]
    q_out[...] = pq[:, 128:]


def _edge_enc_body(f_ref, w1, w2, w3, aux, out):
    a = aux[...]
    x = _relu(_dot(f_ref[...], w1[...]) + a[0:1])
    x = _relu(_dot(x, w2[...]) + a[1:2])
    x = _dot(x, w3[...]) + a[2:3]
    out[...] = _ln(x, a[3:4], a[4:5])


def _edge_step_body(gs, gr, el, w1c, w2, w3, aux, out):
    a = aux[...]
    elv = el[...]
    x = _relu(gs[...] + gr[...] + _dot(elv, w1c[...]))
    x = _relu(_dot(x, w2[...]) + a[0:1])
    x = _dot(x, w3[...]) + a[1:2]
    out[...] = _ln(x, a[2:3], a[3:4]) + elv


def _node_step_body(nl, a0, a1, v1, v2, v3, aux, wpq, out, p_out, q_out):
    a = aux[...]
    nlv = nl[...]
    x = jnp.concatenate([nlv, a0[...] + a1[...]], axis=1)
    x = _relu(_dot(x, v1[...]) + a[0:1])
    x = _relu(_dot(x, v2[...]) + a[1:2])
    x = _dot(x, v3[...]) + a[2:3]
    x = _ln(x, a[3:4], a[4:5]) + nlv
    out[...] = x
    pq = _dot(x, wpq[...])
    p_out[...] = pq[:, :128] + a[5:6]
    q_out[...] = pq[:, 128:]


def _node_last_body(nl, a0, a1, v1, v2, v3, aux, d1, d2, d3, daux, out):
    a = aux[...]
    da = daux[...]
    nlv = nl[...]
    x = jnp.concatenate([nlv, a0[...] + a1[...]], axis=1)
    x = _relu(_dot(x, v1[...]) + a[0:1])
    x = _relu(_dot(x, v2[...]) + a[1:2])
    x = _dot(x, v3[...]) + a[2:3]
    x = _ln(x, a[3:4], a[4:5]) + nlv
    y = _relu(_dot(x, d1[...]) + da[0:1])
    y = _relu(_dot(y, d2[...]) + da[1:2])
    y = _dot(y, d3[...]) + da[2:3, :3]
    out[...] = y


def _pad8(rows):
    out = jnp.zeros((8, 128), F32)
    for i, r in enumerate(rows):
        out = out.at[i, : r.shape[0]].set(r)
    return out


def _call_tc(body, grid, in_arrays, in_specs, out_shapes, out_specs):
    return pl.pallas_call(
        body,
        grid=(grid,),
        in_specs=in_specs,
        out_specs=out_specs,
        out_shape=out_shapes,
        compiler_params=pltpu.CompilerParams(
            dimension_semantics=("arbitrary",),
        ),
    )(*in_arrays)


_NC = 2
_NS = 16
_NW = _NC * _NS
_CH = 128


def _gather_pq(p_tab, q_tab, senders, receivers):
    e, d = p_tab.shape[0], p_tab.shape[1]
    e_edges = senders.shape[0]
    nchunks = e_edges // _CH
    maxiter = (nchunks + _NW - 1) // _NW
    mesh = plsc.VectorSubcoreMesh(core_axis_name="c", subcore_axis_name="s")

    assert maxiter % 2 == 0

    @functools.partial(
        pl.kernel, mesh=mesh,
        out_type=(jax.ShapeDtypeStruct((e_edges, d), p_tab.dtype),
                  jax.ShapeDtypeStruct((e_edges, d), q_tab.dtype)),
        scratch_types=[
            pltpu.VMEM((2, _CH), jnp.int32), pltpu.VMEM((2, _CH), jnp.int32),
            pltpu.VMEM((2, _CH, d), p_tab.dtype), pltpu.VMEM((2, _CH, d), q_tab.dtype),
            pltpu.SemaphoreType.DMA, pltpu.SemaphoreType.DMA,
            pltpu.SemaphoreType.DMA, pltpu.SemaphoreType.DMA,
            pltpu.SemaphoreType.DMA, pltpu.SemaphoreType.DMA,
        ])
    def gk(p_hbm, q_hbm, s_hbm, r_hbm, op_hbm, oq_hbm, sidx, ridx, prow, qrow,
           semi0, semi1, semg0, semg1, semw0, semw1):
        wid = lax.axis_index("c") * _NS + lax.axis_index("s")
        semi = (semi0, semi1)
        semg = (semg0, semg1)
        semw = (semw0, semw1)

        def c_of(j):
            return wid + j * _NW

        def idx_copies(j, s):
            base = c_of(j) * _CH
            return (pltpu.make_async_copy(s_hbm.at[pl.ds(base, _CH)], sidx.at[s], semi[s]),
                    pltpu.make_async_copy(r_hbm.at[pl.ds(base, _CH)], ridx.at[s], semi[s]))

        def gather_copies(j, s):
            del j
            return (pltpu.make_async_copy(p_hbm.at[sidx.at[s]], prow.at[s], semg[s]),
                    pltpu.make_async_copy(q_hbm.at[ridx.at[s]], qrow.at[s], semg[s]))

        def wb_copies(j, s):
            base = c_of(j) * _CH
            return (pltpu.make_async_copy(prow.at[s], op_hbm.at[pl.ds(base, _CH)], semw[s]),
                    pltpu.make_async_copy(qrow.at[s], oq_hbm.at[pl.ds(base, _CH)], semw[s]))

        def start(mk, j, s):
            for cp in mk(j, s):
                cp.start()

        def wait(mk, j, s):
            for cp in mk(j, s):
                cp.wait()

        def guarded(cond, fn, mk, j, s):
            @pl.when(cond)
            def _():
                fn(mk, j, s)

        start(idx_copies, 0, 0)

        def body(k, carry):
            for s in (0, 1):
                j = 2 * k + s
                c = c_of(j)
                not_first = k >= 1 if s == 0 else True
                guarded(jnp.logical_and(k >= 1, c - 2 * _NW < nchunks),
                        wait, wb_copies, j - 2, s)
                guarded(c < nchunks, wait, idx_copies, j, s)
                guarded(c < nchunks, start, gather_copies, j, s)
                guarded(jnp.logical_and(not_first, c - _NW < nchunks),
                        wait, gather_copies, j - 1, 1 - s)
                guarded(jnp.logical_and(not_first, c - _NW < nchunks),
                        start, wb_copies, j - 1, 1 - s)
                guarded(c + _NW < nchunks, start, idx_copies, j + 1, 1 - s)
            return carry

        lax.fori_loop(0, maxiter // 2, body, 0)
        jl = maxiter - 1
        guarded(c_of(jl) < nchunks, wait, gather_copies, jl, jl % 2)
        guarded(c_of(jl) < nchunks, start, wb_copies, jl, jl % 2)
        guarded(c_of(jl - 1) < nchunks, wait, wb_copies, jl - 1, (jl - 1) % 2)
        guarded(c_of(jl) < nchunks, wait, wb_copies, jl, jl % 2)

    return gk(p_tab, q_tab, senders, receivers)


def _scatter_partials(el, receivers, n):
    e_edges, d = el.shape
    nchunks = e_edges // _CH
    maxiter = (nchunks + _NW - 1) // _NW
    rpt = (n // (8 * _NS)) * 8
    rem = n - rpt * _NS
    mesh = plsc.VectorSubcoreMesh(core_axis_name="c", subcore_axis_name="s")

    assert maxiter % 2 == 0

    @functools.partial(
        pl.kernel, mesh=mesh,
        out_type=jax.ShapeDtypeStruct((_NC, n, d), F32),
        scratch_types=[
            pltpu.VMEM((2, _CH), jnp.int32),
            pltpu.VMEM((2, _CH, d), F32),
            pltpu.VMEM_SHARED((n, d), F32),
            pltpu.SemaphoreType.DMA, pltpu.SemaphoreType.DMA,
            pltpu.SemaphoreType.DMA, pltpu.SemaphoreType.DMA,
            pltpu.SemaphoreType.DMA,
        ])
    def sk(el_hbm, r_hbm, z_hbm, out_hbm, ridx, row, acc,
           seml0, seml1, sems0, sems1, semz):
        cid = lax.axis_index("c")
        sid = lax.axis_index("s")
        wid = cid * _NS + sid
        seml = (seml0, seml1)
        sems = (sems0, sems1)

        def c_of(j):
            return wid + j * _NW

        def load_copies(j, s):
            base = c_of(j) * _CH
            return (pltpu.make_async_copy(r_hbm.at[pl.ds(base, _CH)], ridx.at[s], seml[s]),
                    pltpu.make_async_copy(el_hbm.at[pl.ds(base, _CH)], row.at[s], seml[s]))

        def scat_start(s):
            pltpu.async_copy(row.at[s], acc.at[ridx.at[s]], sems[s], add=True)

        def scat_wait(s):
            pltpu.make_async_copy(row.at[s], acc.at[ridx.at[s]], sems[s]).wait()

        def start(mk, j, s):
            for cp in mk(j, s):
                cp.start()

        def wait(mk, j, s):
            for cp in mk(j, s):
                cp.wait()

        def guarded(cond, fn, *args):
            @pl.when(cond)
            def _():
                fn(*args)

        start(load_copies, 0, 0)
        zinit = pltpu.make_async_copy(z_hbm.at[pl.ds(sid * rpt, rpt)],
                                      acc.at[pl.ds(sid * rpt, rpt)], semz)
        zinit.start()
        if rem:
            zrem = pltpu.make_async_copy(z_hbm.at[pl.ds(rpt * _NS, rem)],
                                         acc.at[pl.ds(rpt * _NS, rem)], semz)

            @pl.when(sid == 0)
            def _():
                zrem.start()
                zrem.wait()
        zinit.wait()
        plsc.subcore_barrier()

        def body(k, carry):
            for s in (0, 1):
                j = 2 * k + s
                c = c_of(j)
                not_first = k >= 1 if s == 0 else True
                guarded(c < nchunks, wait, load_copies, j, s)
                guarded(c < nchunks, scat_start, s)
                guarded(jnp.logical_and(not_first, c - _NW < nchunks),
                        scat_wait, 1 - s)
                guarded(c + _NW < nchunks, start, load_copies, j + 1, 1 - s)
            return carry

        lax.fori_loop(0, maxiter // 2, body, 0)
        jl = maxiter - 1
        guarded(c_of(jl) < nchunks, scat_wait, jl % 2)
        plsc.subcore_barrier()
        pltpu.sync_copy(acc.at[pl.ds(sid * rpt, rpt)],
                        out_hbm.at[cid, pl.ds(sid * rpt, rpt)])
        if rem:
            @pl.when(sid == 0)
            def _():
                pltpu.sync_copy(acc.at[pl.ds(rpt * _NS, rem)],
                                out_hbm.at[cid, pl.ds(rpt * _NS, rem)])

    out = sk(el, receivers, jnp.zeros((n, d), F32))
    return out[0], out[1]


def _normalize_feat(x, eps=1e-8):
    mean = jnp.mean(x, axis=0, keepdims=True)
    second = jnp.mean(x * x, axis=0, keepdims=True)
    std = jnp.sqrt(jnp.maximum(second - mean * mean, 0.0))
    return (x - mean) / jnp.maximum(std, eps)


def _safe_norm2(x):
    s = jnp.sum(x * x, axis=-1, keepdims=True)
    out = jnp.sqrt(jnp.where(s > 0, s, 1.0))
    return jnp.where(s > 0, out, 0.0)


def kernel(node_type, pressure, target_pressure, mesh_pos, world_pos, senders,
           receivers, is_trainning, params):
    n = node_type.shape[0]
    e = senders.shape[0]
    steps = len(params["blocks"])

    pressure_increase = target_pressure - pressure
    expanded = jnp.broadcast_to(pressure_increase, (n, 1))
    oh = jax.nn.one_hot(node_type[:, 0], 9, dtype=F32)
    node_features = _normalize_feat(jnp.concatenate([expanded, oh], axis=-1))
    rel_w = jnp.take(world_pos, senders, axis=0) - jnp.take(world_pos, receivers, axis=0)
    rel_m = jnp.take(mesh_pos, senders, axis=0) - jnp.take(mesh_pos, receivers, axis=0)
    edge_features = _normalize_feat(
        jnp.concatenate([rel_w, _safe_norm2(rel_w), rel_m, _safe_norm2(rel_m)], axis=-1))

    blocks = params["blocks"]
    ew = []
    for t in range(steps):
        em = blocks[t]["edge_mlp"]
        w1 = em["Ws"][0]
        ew.append({
            "w1a": w1[:128], "w1b": w1[128:256], "w1c": w1[256:384],
            "b1": em["bs"][0], "w2": em["Ws"][1], "w3": em["Ws"][2],
            "aux": _pad8([em["bs"][1], em["bs"][2], em["ln_s"], em["ln_b"]]),
        })
        ew[t]["wpq"] = jnp.concatenate([ew[t]["w1a"], ew[t]["w1b"]], axis=1)

    nw = []
    for t in range(steps):
        nm = blocks[t]["node_mlp"]
        nxt = ew[t + 1] if t + 1 < steps else None
        nw.append({
            "v1": nm["Ws"][0], "v2": nm["Ws"][1], "v3": nm["Ws"][2],
            "aux": _pad8([nm["bs"][0], nm["bs"][1], nm["bs"][2], nm["ln_s"], nm["ln_b"]]
                         + ([nxt["b1"]] if nxt is not None else [])),
        })

    ne_grid = n // BN
    ee_grid = e // BE

    enc = params["node_enc"]
    enc_aux = _pad8([enc["bs"][0], enc["bs"][1], enc["bs"][2], enc["ln_s"],
                     enc["ln_b"], ew[0]["b1"]])
    x1n = jax.nn.relu(node_features @ enc["Ws"][0] + enc["bs"][0])
    nl, p_tab, q_tab = _call_tc(
        _encoder_body, ne_grid,
        [x1n, enc["Ws"][1], enc["Ws"][2], enc_aux, ew[0]["wpq"]],
        [_rows(BN), _full((128, 128)), _full((128, 128)),
         _full((8, 128)), _full((128, 256))],
        (jax.ShapeDtypeStruct((n, 128), F32),) * 3,
        (_rows(BN),) * 3,
    )

    eenc = params["edge_enc"]
    eenc_aux = _pad8([eenc["bs"][0], eenc["bs"][1], eenc["bs"][2], eenc["ln_s"], eenc["ln_b"]])
    ef = edge_features.shape[1]
    el = _call_tc(
        _edge_enc_body, ee_grid,
        [edge_features, eenc["Ws"][0], eenc["Ws"][1], eenc["Ws"][2], eenc_aux],
        [_rows(BE, ef), _full((ef, 128)), _full((128, 128)), _full((128, 128)), _full((8, 128))],
        jax.ShapeDtypeStruct((e, 128), F32),
        _rows(BE),
    )

    for t in range(steps):
        gs, gr = _gather_pq(p_tab, q_tab, senders, receivers)
        el = _call_tc(
            _edge_step_body, ee_grid,
            [gs, gr, el, ew[t]["w1c"], ew[t]["w2"], ew[t]["w3"], ew[t]["aux"]],
            [_rows(BE)] * 3 + [_full((128, 128))] * 3 + [_full((8, 128))],
            jax.ShapeDtypeStruct((e, 128), F32),
            _rows(BE),
        )
        a0, a1 = _scatter_partials(el, receivers, n)
        if t + 1 < steps:
            nl, p_tab, q_tab = _call_tc(
                _node_step_body, ne_grid,
                [nl, a0, a1, nw[t]["v1"], nw[t]["v2"], nw[t]["v3"], nw[t]["aux"],
                 ew[t + 1]["wpq"]],
                [_rows(BN)] * 3 + [_full((256, 128)), _full((128, 128)), _full((128, 128)),
                                   _full((8, 128)), _full((128, 256))],
                (jax.ShapeDtypeStruct((n, 128), F32),) * 3,
                (_rows(BN),) * 3,
            )
        else:
            dec = params["decoder"]
            daux = _pad8([dec["bs"][0], dec["bs"][1],
                          jnp.pad(dec["bs"][2], (0, 125))])
            out = _call_tc(
                _node_last_body, ne_grid,
                [nl, a0, a1, nw[t]["v1"], nw[t]["v2"], nw[t]["v3"], nw[t]["aux"],
                 dec["Ws"][0], dec["Ws"][1], dec["Ws"][2], daux],
                [_rows(BN)] * 3 + [_full((256, 128)), _full((128, 128)), _full((128, 128)),
                                   _full((8, 128)), _full((128, 128)), _full((128, 128)),
                                   _full((128, 3)), _full((8, 128))],
                jax.ShapeDtypeStruct((n, 3), F32),
                _rows(BN, 3),
            )
    return out
